# Initial kernel scaffold; baseline (speedup 1.0000x reference)
import jax
import jax.numpy as jnp
from jax.experimental import pallas as pl


def kernel(x_column, x_constraint, edge_serves, edge_served_by, Wcol, bcol, Wcon, bcon, c1_cs_Wl, c1_cs_bl, c1_cs_Wr, c1_sb_Wl, c1_sb_bl, c1_sb_Wr, c2_cs_Wl, c2_cs_bl, c2_cs_Wr, c2_sb_Wl, c2_sb_bl, c2_sb_Wr, qW1, qb1, qW2, qb2):
    raise NotImplementedError("write your pallas kernel here")


# speedup vs baseline: 1.0655x; 1.0655x over previous
import functools

import jax
import jax.numpy as jnp
from jax import lax
from jax.experimental import pallas as pl
from jax.experimental.pallas import tpu as pltpu

N = 50000
H = 64
HH = 32
BN = 1000
GRID = N // BN


def _proj_body(xcol_ref, xcon_ref, wcolT_ref, bcol_ref, wconT_ref, bcon_ref,
               xc_ref, xn_ref):
    xc = jnp.maximum(
        jnp.dot(xcol_ref[...], wcolT_ref[...],
                preferred_element_type=jnp.float32) + bcol_ref[...], 0.0)
    xn = jnp.maximum(
        jnp.dot(xcon_ref[...], wconT_ref[...],
                preferred_element_type=jnp.float32) + bcon_ref[...], 0.0)
    xc_ref[0] = xc[:, :HH]
    xc_ref[1] = xc[:, HH:]
    xn_ref[0] = xn[:, :HH]
    xn_ref[1] = xn[:, HH:]


def _proj(x_column, x_constraint, Wcol, bcol, Wcon, bcon):
    return pl.pallas_call(
        _proj_body,
        grid=(GRID,),
        in_specs=[
            pl.BlockSpec((BN, 128), lambda i: (i, 0)),
            pl.BlockSpec((BN, 64), lambda i: (i, 0)),
            pl.BlockSpec((128, 64), lambda i: (0, 0)),
            pl.BlockSpec((1, 64), lambda i: (0, 0)),
            pl.BlockSpec((64, 64), lambda i: (0, 0)),
            pl.BlockSpec((1, 64), lambda i: (0, 0)),
        ],
        out_specs=[
            pl.BlockSpec((2, BN, HH), lambda i: (0, i, 0)),
            pl.BlockSpec((2, BN, HH), lambda i: (0, i, 0)),
        ],
        out_shape=[
            jax.ShapeDtypeStruct((2, N, HH), jnp.float32),
            jax.ShapeDtypeStruct((2, N, HH), jnp.float32),
        ],
    )(x_column, x_constraint, Wcol.T, bcol.reshape(1, 64),
      Wcon.T, bcon.reshape(1, 64))


def _conv_body(split_out, head, *refs):
    if head:
        (s_ref, cnt_ref, xd_ref, wlT_ref, bl_ref, wrT_ref,
         qW1T_ref, qb1_ref, qW2T_ref, qb2_ref, out_ref) = refs
    else:
        s_ref, cnt_ref, xd_ref, wlT_ref, bl_ref, wrT_ref, out_ref = refs
    s = jnp.concatenate([s_ref[0], s_ref[1]], axis=1)
    inv = 1.0 / jnp.maximum(cnt_ref[0], 1.0)
    mean = s * inv
    if xd_ref.shape[0] == 2:
        xd = jnp.concatenate([xd_ref[0], xd_ref[1]], axis=1)
    else:
        xd = xd_ref[...]
    out = jnp.maximum(
        jnp.dot(mean, wlT_ref[...], preferred_element_type=jnp.float32)
        + bl_ref[...]
        + jnp.dot(xd, wrT_ref[...], preferred_element_type=jnp.float32), 0.0)
    if head:
        h = jnp.maximum(
            jnp.dot(out, qW1T_ref[...], preferred_element_type=jnp.float32)
            + qb1_ref[...], 0.0)
        q = jnp.dot(h, qW2T_ref[...], preferred_element_type=jnp.float32) \
            + qb2_ref[...]
        out_ref[...] = q
    elif split_out:
        out_ref[0] = out[:, :HH]
        out_ref[1] = out[:, HH:]
    else:
        out_ref[...] = out


def _conv(summed, cnt, xd, Wl, bl, Wr, split_out=False, head_w=None):
    head = head_w is not None
    xd_split = (xd.ndim == 3)
    in_specs = [
        pl.BlockSpec((2, BN, HH), lambda i: (0, i, 0)),
        pl.BlockSpec((1, BN, 1), lambda i: (0, i, 0)),
        (pl.BlockSpec((2, BN, HH), lambda i: (0, i, 0)) if xd_split
         else pl.BlockSpec((BN, 64), lambda i: (i, 0))),
        pl.BlockSpec((64, 64), lambda i: (0, 0)),
        pl.BlockSpec((1, 64), lambda i: (0, 0)),
        pl.BlockSpec((64, 64), lambda i: (0, 0)),
    ]
    args = [summed, cnt.reshape(2, N, 1), xd, Wl.T, bl.reshape(1, 64), Wr.T]
    if head:
        qW1, qb1, qW2, qb2 = head_w
        in_specs += [
            pl.BlockSpec((64, 32), lambda i: (0, 0)),
            pl.BlockSpec((1, 32), lambda i: (0, 0)),
            pl.BlockSpec((32, 1), lambda i: (0, 0)),
            pl.BlockSpec((1, 1), lambda i: (0, 0)),
        ]
        args += [qW1.T, qb1.reshape(1, 32), qW2.T, qb2.reshape(1, 1)]
        out_spec = pl.BlockSpec((BN, 1), lambda i: (i, 0))
        out_shape = jax.ShapeDtypeStruct((N, 1), jnp.float32)
    elif split_out:
        out_spec = pl.BlockSpec((2, BN, HH), lambda i: (0, i, 0))
        out_shape = jax.ShapeDtypeStruct((2, N, HH), jnp.float32)
    else:
        out_spec = pl.BlockSpec((BN, 64), lambda i: (i, 0))
        out_shape = jax.ShapeDtypeStruct((N, 64), jnp.float32)
    return pl.pallas_call(
        functools.partial(_conv_body, split_out, head),
        grid=(GRID,),
        in_specs=in_specs,
        out_specs=out_spec,
        out_shape=out_shape,
    )(*args)


def _spmm_jax(x_split, src, dst, with_cnt):
    x64 = jnp.concatenate([x_split[0], x_split[1]], axis=1)
    msgs = jnp.take(x64, src, axis=0)
    summed = jax.ops.segment_sum(msgs, dst, num_segments=N)
    s_split = jnp.stack([summed[:, :HH], summed[:, HH:]])
    if not with_cnt:
        return s_split, None
    cnt = jax.ops.segment_sum(jnp.ones((src.shape[0],), jnp.float32), dst,
                              num_segments=N)
    return s_split, jnp.stack([cnt, cnt])


def kernel(x_column, x_constraint, edge_serves, edge_served_by,
           Wcol, bcol, Wcon, bcon,
           c1_cs_Wl, c1_cs_bl, c1_cs_Wr, c1_sb_Wl, c1_sb_bl, c1_sb_Wr,
           c2_cs_Wl, c2_cs_bl, c2_cs_Wr, c2_sb_Wl, c2_sb_bl, c2_sb_Wr,
           qW1, qb1, qW2, qb2):
    s_cs, d_cs = edge_serves[0], edge_serves[1]
    s_sb, d_sb = edge_served_by[0], edge_served_by[1]

    xc, xn = _proj(x_column, x_constraint, Wcol, bcol, Wcon, bcon)

    sum_cs, cnt_cs = _spmm_jax(xc, s_cs, d_cs, True)
    sum_sb, cnt_sb = _spmm_jax(xn, s_sb, d_sb, True)

    xn1 = _conv(sum_cs, cnt_cs, xn, c1_cs_Wl, c1_cs_bl, c1_cs_Wr,
                split_out=True)
    xc1 = _conv(sum_sb, cnt_sb, xc, c1_sb_Wl, c1_sb_bl, c1_sb_Wr)

    sum_sb2, _ = _spmm_jax(xn1, s_sb, d_sb, False)

    q = _conv(sum_sb2, cnt_sb, xc1, c2_sb_Wl, c2_sb_bl, c2_sb_Wr,
              head_w=(qW1, qb1, qW2, qb2))
    return q.reshape(-1)


# speedup vs baseline: 8.8056x; 8.2646x over previous
import functools

import jax
import jax.numpy as jnp
from jax import lax
from jax.experimental import pallas as pl
from jax.experimental.pallas import tpu as pltpu

N = 50000
H = 64
HH = 32
BN = 1000
GRID = N // BN


def _proj_body(xcol_ref, xcon_ref, wcolT_ref, bcol_ref, wconT_ref, bcon_ref,
               xc_ref, xn_ref):
    xc = jnp.maximum(
        jnp.dot(xcol_ref[...], wcolT_ref[...],
                preferred_element_type=jnp.float32) + bcol_ref[...], 0.0)
    xn = jnp.maximum(
        jnp.dot(xcon_ref[...], wconT_ref[...],
                preferred_element_type=jnp.float32) + bcon_ref[...], 0.0)
    xc_ref[0] = xc[:, :HH]
    xc_ref[1] = xc[:, HH:]
    xn_ref[0] = xn[:, :HH]
    xn_ref[1] = xn[:, HH:]


def _proj(x_column, x_constraint, Wcol, bcol, Wcon, bcon):
    return pl.pallas_call(
        _proj_body,
        grid=(GRID,),
        in_specs=[
            pl.BlockSpec((BN, 128), lambda i: (i, 0)),
            pl.BlockSpec((BN, 64), lambda i: (i, 0)),
            pl.BlockSpec((128, 64), lambda i: (0, 0)),
            pl.BlockSpec((1, 64), lambda i: (0, 0)),
            pl.BlockSpec((64, 64), lambda i: (0, 0)),
            pl.BlockSpec((1, 64), lambda i: (0, 0)),
        ],
        out_specs=[
            pl.BlockSpec((2, BN, HH), lambda i: (0, i, 0)),
            pl.BlockSpec((2, BN, HH), lambda i: (0, i, 0)),
        ],
        out_shape=[
            jax.ShapeDtypeStruct((2, N, HH), jnp.float32),
            jax.ShapeDtypeStruct((2, N, HH), jnp.float32),
        ],
    )(x_column, x_constraint, Wcol.T, bcol.reshape(1, 64),
      Wcon.T, bcon.reshape(1, 64))


def _conv_body(split_out, head, *refs):
    if head:
        (s_ref, cnt_ref, xd_ref, wlT_ref, bl_ref, wrT_ref,
         qW1T_ref, qb1_ref, qW2T_ref, qb2_ref, out_ref) = refs
    else:
        s_ref, cnt_ref, xd_ref, wlT_ref, bl_ref, wrT_ref, out_ref = refs
    s = jnp.concatenate([s_ref[0], s_ref[1]], axis=1)
    inv = 1.0 / jnp.maximum(cnt_ref[0], 1.0)
    mean = s * inv
    if xd_ref.shape[0] == 2:
        xd = jnp.concatenate([xd_ref[0], xd_ref[1]], axis=1)
    else:
        xd = xd_ref[...]
    out = jnp.maximum(
        jnp.dot(mean, wlT_ref[...], preferred_element_type=jnp.float32)
        + bl_ref[...]
        + jnp.dot(xd, wrT_ref[...], preferred_element_type=jnp.float32), 0.0)
    if head:
        h = jnp.maximum(
            jnp.dot(out, qW1T_ref[...], preferred_element_type=jnp.float32)
            + qb1_ref[...], 0.0)
        q = jnp.dot(h, qW2T_ref[...], preferred_element_type=jnp.float32) \
            + qb2_ref[...]
        out_ref[...] = q
    elif split_out:
        out_ref[0] = out[:, :HH]
        out_ref[1] = out[:, HH:]
    else:
        out_ref[...] = out


def _conv(summed, cnt, xd, Wl, bl, Wr, split_out=False, head_w=None):
    head = head_w is not None
    xd_split = (xd.ndim == 3)
    in_specs = [
        pl.BlockSpec((2, BN, HH), lambda i: (0, i, 0)),
        pl.BlockSpec((1, BN, 1), lambda i: (0, i, 0)),
        (pl.BlockSpec((2, BN, HH), lambda i: (0, i, 0)) if xd_split
         else pl.BlockSpec((BN, 64), lambda i: (i, 0))),
        pl.BlockSpec((64, 64), lambda i: (0, 0)),
        pl.BlockSpec((1, 64), lambda i: (0, 0)),
        pl.BlockSpec((64, 64), lambda i: (0, 0)),
    ]
    args = [summed, cnt.reshape(2, N, 1), xd, Wl.T, bl.reshape(1, 64), Wr.T]
    if head:
        qW1, qb1, qW2, qb2 = head_w
        in_specs += [
            pl.BlockSpec((64, 32), lambda i: (0, 0)),
            pl.BlockSpec((1, 32), lambda i: (0, 0)),
            pl.BlockSpec((32, 1), lambda i: (0, 0)),
            pl.BlockSpec((1, 1), lambda i: (0, 0)),
        ]
        args += [qW1.T, qb1.reshape(1, 32), qW2.T, qb2.reshape(1, 1)]
        out_spec = pl.BlockSpec((BN, 1), lambda i: (i, 0))
        out_shape = jax.ShapeDtypeStruct((N, 1), jnp.float32)
    elif split_out:
        out_spec = pl.BlockSpec((2, BN, HH), lambda i: (0, i, 0))
        out_shape = jax.ShapeDtypeStruct((2, N, HH), jnp.float32)
    else:
        out_spec = pl.BlockSpec((BN, 64), lambda i: (i, 0))
        out_shape = jax.ShapeDtypeStruct((N, 64), jnp.float32)
    return pl.pallas_call(
        functools.partial(_conv_body, split_out, head),
        grid=(GRID,),
        in_specs=in_specs,
        out_specs=out_spec,
        out_shape=out_shape,
    )(*args)


E_TOT = 800000
N_SUB = 16
EPS = 51200
E_PAD = EPS - E_TOT // N_SUB
CH = 128
KB = 4
BLOCKS = EPS // (CH * KB)
NDUM = 240
ACC_ROWS = N + NDUM
ZR = 160
WB = 3200

from jax.experimental.pallas import tpu_sc as plsc


def _make_spmm(with_cnt):
    mesh = plsc.VectorSubcoreMesh(core_axis_name="c", subcore_axis_name="s")
    out_type = [jax.ShapeDtypeStruct((2, N, HH), jnp.float32)]
    if with_cnt:
        out_type.append(jax.ShapeDtypeStruct((2, N), jnp.float32))
    scratch = [
        pltpu.VMEM_SHARED((ACC_ROWS, HH), jnp.float32),
        pltpu.VMEM((KB, CH), jnp.int32),
        pltpu.VMEM((KB, CH), jnp.int32),
        pltpu.VMEM((KB * CH, HH), jnp.float32),
        pltpu.SemaphoreType.DMA,
    ]
    if with_cnt:
        scratch.insert(1, pltpu.VMEM_SHARED((ACC_ROWS,), jnp.float32))
        scratch.append(pltpu.VMEM((CH,), jnp.float32))
        scratch.append(pltpu.VMEM((ZR,), jnp.float32))

    @functools.partial(
        pl.kernel, mesh=mesh, out_type=out_type, scratch_types=scratch,
        compiler_params=pltpu.CompilerParams(use_tc_tiling_on_sc=False))
    def spmm(table, srcs, dsts, *rest):
        if with_cnt:
            (out, cnt_out, acc, cnt_acc, srcb, dstb, rows, sem,
             ones, zrow1) = rest
        else:
            out, acc, srcb, dstb, rows, sem = rest
        c = lax.axis_index("c")
        s = lax.axis_index("s")

        def zbody(i, carry):
            z16 = jnp.zeros((16,), jnp.float32)
            rows[i, pl.ds(0, 16)] = z16
            rows[i, pl.ds(16, 16)] = z16
            return carry
        lax.fori_loop(0, ZR, zbody, 0)
        if with_cnt:
            def z1body(i, carry):
                zrow1[pl.ds(i * 16, 16)] = jnp.zeros((16,), jnp.float32)
                return carry
            lax.fori_loop(0, ZR // 16, z1body, 0)
            for k in range(CH // 16):
                ones[pl.ds(k * 16, 16)] = jnp.ones((16,), jnp.float32)

        base0 = s * WB
        ncop = jnp.where(s < N_SUB - 1, WB // ZR, (ACC_ROWS - 15 * WB) // ZR)
        def zacc(kk, carry):
            off = base0 + kk * ZR
            pltpu.sync_copy(rows.at[pl.ds(0, ZR)], acc.at[pl.ds(off, ZR)])
            if with_cnt:
                pltpu.sync_copy(zrow1, cnt_acc.at[pl.ds(off, ZR)])
            return carry
        lax.fori_loop(0, ncop, zacc, 0)
        plsc.subcore_barrier()

        def blk_body(blk, carry):
            gblk = s * BLOCKS + blk
            pltpu.sync_copy(srcs.at[c, gblk], srcb)
            pltpu.sync_copy(dsts.at[gblk], dstb)
            cps = []
            for j in range(KB):
                cps.append(pltpu.async_copy(
                    table.at[srcb.at[j]],
                    rows.at[pl.ds(j * CH, CH)], sem))
            for j in range(KB):
                cps[j].wait()
            for j in range(KB):
                pltpu.sync_copy(rows.at[pl.ds(j * CH, CH)],
                                acc.at[dstb.at[j]], add=True)
                if with_cnt:
                    pltpu.sync_copy(ones, cnt_acc.at[dstb.at[j]], add=True)
            return carry
        lax.fori_loop(0, BLOCKS, blk_body, 0)
        plsc.subcore_barrier()

        @pl.when(s < N_SUB - 1)
        def _():
            pltpu.sync_copy(acc.at[pl.ds(base0, WB)],
                            out.at[c, pl.ds(base0, WB)])
            if with_cnt:
                pltpu.sync_copy(cnt_acc.at[pl.ds(base0, WB)],
                                cnt_out.at[c, pl.ds(base0, WB)])

        @pl.when(s == N_SUB - 1)
        def _():
            nlast = N - (N_SUB - 1) * WB
            pltpu.sync_copy(acc.at[pl.ds((N_SUB - 1) * WB, nlast)],
                            out.at[c, pl.ds((N_SUB - 1) * WB, nlast)])
            if with_cnt:
                pltpu.sync_copy(cnt_acc.at[pl.ds((N_SUB - 1) * WB, nlast)],
                                cnt_out.at[c, pl.ds((N_SUB - 1) * WB, nlast)])

    return spmm


_spmm_cnt = _make_spmm(True)
_spmm_nocnt = _make_spmm(False)


def _prep_edges(src, dst):
    per = E_TOT // N_SUB
    src_r = src.reshape(N_SUB, per)
    pad_src = ((jnp.arange(N_SUB * E_PAD, dtype=jnp.int32) * 97) % N) \
        .reshape(N_SUB, E_PAD)
    srcp = jnp.concatenate([src_r, pad_src], axis=1) \
        .reshape(N_SUB * BLOCKS, KB, CH)
    srcs = jnp.stack([srcp, srcp + N])
    dst_r = dst.reshape(N_SUB, per)
    pad_dst = (N + (jnp.arange(N_SUB * E_PAD, dtype=jnp.int32) % NDUM)) \
        .reshape(N_SUB, E_PAD)
    dstp = jnp.concatenate([dst_r, pad_dst], axis=1) \
        .reshape(N_SUB * BLOCKS, KB, CH)
    return srcs, dstp


def _spmm_sc(x_split, srcs, dsts, with_cnt):
    table = x_split.reshape(2 * N, HH)
    if with_cnt:
        out, cnt = _spmm_cnt(table, srcs, dsts)
        return out, cnt
    (out,) = _spmm_nocnt(table, srcs, dsts)
    return out, None


def kernel(x_column, x_constraint, edge_serves, edge_served_by,
           Wcol, bcol, Wcon, bcon,
           c1_cs_Wl, c1_cs_bl, c1_cs_Wr, c1_sb_Wl, c1_sb_bl, c1_sb_Wr,
           c2_cs_Wl, c2_cs_bl, c2_cs_Wr, c2_sb_Wl, c2_sb_bl, c2_sb_Wr,
           qW1, qb1, qW2, qb2):
    s_cs, d_cs = edge_serves[0], edge_serves[1]
    s_sb, d_sb = edge_served_by[0], edge_served_by[1]

    xc, xn = _proj(x_column, x_constraint, Wcol, bcol, Wcon, bcon)

    cs_srcs, cs_dsts = _prep_edges(s_cs, d_cs)
    sb_srcs, sb_dsts = _prep_edges(s_sb, d_sb)

    sum_cs, cnt_cs = _spmm_sc(xc, cs_srcs, cs_dsts, True)
    sum_sb, cnt_sb = _spmm_sc(xn, sb_srcs, sb_dsts, True)

    xn1 = _conv(sum_cs, cnt_cs, xn, c1_cs_Wl, c1_cs_bl, c1_cs_Wr,
                split_out=True)
    xc1 = _conv(sum_sb, cnt_sb, xc, c1_sb_Wl, c1_sb_bl, c1_sb_Wr)

    sum_sb2, _ = _spmm_sc(xn1, sb_srcs, sb_dsts, False)

    q = _conv(sum_sb2, cnt_sb, xc1, c2_sb_Wl, c2_sb_bl, c2_sb_Wr,
              head_w=(qW1, qb1, qW2, qb2))
    return q.reshape(-1)


# speedup vs baseline: 13.5638x; 1.5404x over previous
import functools

import jax
import jax.numpy as jnp
from jax import lax
from jax.experimental import pallas as pl
from jax.experimental.pallas import tpu as pltpu

N = 50000
H = 64
HH = 32
BN = 1000
GRID = N // BN


def _proj_body(xcol_ref, xcon_ref, wcolT_ref, bcol_ref, wconT_ref, bcon_ref,
               xc_ref, xn_ref):
    xc = jnp.maximum(
        jnp.dot(xcol_ref[...], wcolT_ref[...],
                preferred_element_type=jnp.float32) + bcol_ref[...], 0.0)
    xn = jnp.maximum(
        jnp.dot(xcon_ref[...], wconT_ref[...],
                preferred_element_type=jnp.float32) + bcon_ref[...], 0.0)
    xc_ref[0] = xc[:, :HH]
    xc_ref[1] = xc[:, HH:]
    xn_ref[0] = xn[:, :HH]
    xn_ref[1] = xn[:, HH:]


def _proj(x_column, x_constraint, Wcol, bcol, Wcon, bcon):
    return pl.pallas_call(
        _proj_body,
        grid=(GRID,),
        in_specs=[
            pl.BlockSpec((BN, 128), lambda i: (i, 0)),
            pl.BlockSpec((BN, 64), lambda i: (i, 0)),
            pl.BlockSpec((128, 64), lambda i: (0, 0)),
            pl.BlockSpec((1, 64), lambda i: (0, 0)),
            pl.BlockSpec((64, 64), lambda i: (0, 0)),
            pl.BlockSpec((1, 64), lambda i: (0, 0)),
        ],
        out_specs=[
            pl.BlockSpec((2, BN, HH), lambda i: (0, i, 0)),
            pl.BlockSpec((2, BN, HH), lambda i: (0, i, 0)),
        ],
        out_shape=[
            jax.ShapeDtypeStruct((2, N, HH), jnp.float32),
            jax.ShapeDtypeStruct((2, N, HH), jnp.float32),
        ],
    )(x_column, x_constraint, Wcol.T, bcol.reshape(1, 64),
      Wcon.T, bcon.reshape(1, 64))


def _conv_body(split_out, head, *refs):
    if head:
        (s_ref, cnt_ref, xd_ref, wlT_ref, bl_ref, wrT_ref,
         qW1T_ref, qb1_ref, qW2T_ref, qb2_ref, out_ref) = refs
    else:
        s_ref, cnt_ref, xd_ref, wlT_ref, bl_ref, wrT_ref, out_ref = refs
    s = jnp.concatenate([s_ref[0], s_ref[1]], axis=1)
    inv = 1.0 / jnp.maximum(cnt_ref[0], 1.0)
    mean = s * inv
    if xd_ref.shape[0] == 2:
        xd = jnp.concatenate([xd_ref[0], xd_ref[1]], axis=1)
    else:
        xd = xd_ref[...]
    out = jnp.maximum(
        jnp.dot(mean, wlT_ref[...], preferred_element_type=jnp.float32)
        + bl_ref[...]
        + jnp.dot(xd, wrT_ref[...], preferred_element_type=jnp.float32), 0.0)
    if head:
        h = jnp.maximum(
            jnp.dot(out, qW1T_ref[...], preferred_element_type=jnp.float32)
            + qb1_ref[...], 0.0)
        q = jnp.dot(h, qW2T_ref[...], preferred_element_type=jnp.float32) \
            + qb2_ref[...]
        out_ref[...] = q
    elif split_out:
        out_ref[0] = out[:, :HH]
        out_ref[1] = out[:, HH:]
    else:
        out_ref[...] = out


def _conv(summed, cnt, xd, Wl, bl, Wr, split_out=False, head_w=None):
    head = head_w is not None
    xd_split = (xd.ndim == 3)
    in_specs = [
        pl.BlockSpec((2, BN, HH), lambda i: (0, i, 0)),
        pl.BlockSpec((1, BN, 1), lambda i: (0, i, 0)),
        (pl.BlockSpec((2, BN, HH), lambda i: (0, i, 0)) if xd_split
         else pl.BlockSpec((BN, 64), lambda i: (i, 0))),
        pl.BlockSpec((64, 64), lambda i: (0, 0)),
        pl.BlockSpec((1, 64), lambda i: (0, 0)),
        pl.BlockSpec((64, 64), lambda i: (0, 0)),
    ]
    args = [summed, cnt.reshape(2, N, 1), xd, Wl.T, bl.reshape(1, 64), Wr.T]
    if head:
        qW1, qb1, qW2, qb2 = head_w
        in_specs += [
            pl.BlockSpec((64, 32), lambda i: (0, 0)),
            pl.BlockSpec((1, 32), lambda i: (0, 0)),
            pl.BlockSpec((32, 1), lambda i: (0, 0)),
            pl.BlockSpec((1, 1), lambda i: (0, 0)),
        ]
        args += [qW1.T, qb1.reshape(1, 32), qW2.T, qb2.reshape(1, 1)]
        out_spec = pl.BlockSpec((BN, 1), lambda i: (i, 0))
        out_shape = jax.ShapeDtypeStruct((N, 1), jnp.float32)
    elif split_out:
        out_spec = pl.BlockSpec((2, BN, HH), lambda i: (0, i, 0))
        out_shape = jax.ShapeDtypeStruct((2, N, HH), jnp.float32)
    else:
        out_spec = pl.BlockSpec((BN, 64), lambda i: (i, 0))
        out_shape = jax.ShapeDtypeStruct((N, 64), jnp.float32)
    return pl.pallas_call(
        functools.partial(_conv_body, split_out, head),
        grid=(GRID,),
        in_specs=in_specs,
        out_specs=out_spec,
        out_shape=out_shape,
    )(*args)


E_TOT = 800000
N_SUB = 16
EPS = 51200
E_PAD = EPS - E_TOT // N_SUB
CH = 128
KB = 2
BLOCKS = EPS // (CH * KB)
NDUM = 400
ACC_ROWS = N + NDUM
ZR = 200
WB = 3200

from jax.experimental.pallas import tpu_sc as plsc


def _make_spmm(with_cnt):
    mesh = plsc.VectorSubcoreMesh(core_axis_name="c", subcore_axis_name="s")
    out_type = [jax.ShapeDtypeStruct((2, N, HH), jnp.float32)]
    if with_cnt:
        out_type.append(jax.ShapeDtypeStruct((2, N), jnp.float32))
    scratch = [
        pltpu.VMEM_SHARED((ACC_ROWS, HH), jnp.float32),
        pltpu.VMEM((4, KB, CH), jnp.int32),
        pltpu.VMEM((4, KB, CH), jnp.int32),
        pltpu.VMEM((2, KB * CH, HH), jnp.float32),
        pltpu.SemaphoreType.DMA,
        pltpu.SemaphoreType.DMA,
        pltpu.SemaphoreType.DMA,
        pltpu.SemaphoreType.DMA,
    ]
    if with_cnt:
        scratch.insert(1, pltpu.VMEM_SHARED((ACC_ROWS,), jnp.float32))
        scratch.append(pltpu.VMEM((CH,), jnp.float32))
        scratch.append(pltpu.VMEM((ZR,), jnp.float32))
        scratch.append(pltpu.SemaphoreType.DMA)

    @functools.partial(
        pl.kernel, mesh=mesh, out_type=out_type, scratch_types=scratch,
        compiler_params=pltpu.CompilerParams(use_tc_tiling_on_sc=False))
    def spmm(table, srcs, dsts, *rest):
        if with_cnt:
            (out, cnt_out, acc, cnt_acc, srcb, dstb, rows, sem_g0, sem_g1,
             sem_s, sem_i, ones, zrow1, sem_c) = rest
        else:
            (out, acc, srcb, dstb, rows, sem_g0, sem_g1, sem_s,
             sem_i) = rest
        sem_g = (sem_g0, sem_g1)
        c = lax.axis_index("c")
        s = lax.axis_index("s")

        def zbody(i, carry):
            z16 = jnp.zeros((16,), jnp.float32)
            rows[0, i, pl.ds(0, 16)] = z16
            rows[0, i, pl.ds(16, 16)] = z16
            return carry
        lax.fori_loop(0, ZR, zbody, 0)
        if with_cnt:
            def z1body(i, carry):
                zrow1[pl.ds(i * 16, 16)] = jnp.zeros((16,), jnp.float32)
                return carry
            lax.fori_loop(0, ZR // 16, z1body, 0)
            for k in range(CH // 16):
                ones[pl.ds(k * 16, 16)] = jnp.ones((16,), jnp.float32)

        base0 = s * WB
        ncop = jnp.where(s < N_SUB - 1, WB // ZR, (ACC_ROWS - 15 * WB) // ZR)
        def zacc(kk, carry):
            off = base0 + kk * ZR
            pltpu.sync_copy(rows.at[0, pl.ds(0, ZR)], acc.at[pl.ds(off, ZR)])
            if with_cnt:
                pltpu.sync_copy(zrow1, cnt_acc.at[pl.ds(off, ZR)])
            return carry
        lax.fori_loop(0, ncop, zacc, 0)
        plsc.subcore_barrier()

        gbase = s * BLOCKS

        def _fire_gathers(slot, rp):
            for j in range(KB):
                pltpu.async_copy(table.at[srcb.at[slot, j]],
                                 rows.at[rp, pl.ds(j * CH, CH)], sem_g[rp])

        def _drain_gathers(slot, rp):
            for j in range(KB):
                pltpu.make_async_copy(
                    table.at[srcb.at[slot, j]],
                    rows.at[rp, pl.ds(j * CH, CH)], sem_g[rp]).wait()

        def _issue_scatters(slot, rp):
            for j in range(KB):
                pltpu.async_copy(rows.at[rp, pl.ds(j * CH, CH)],
                                 acc.at[dstb.at[slot, j]], sem_s, add=True)
                if with_cnt:
                    pltpu.async_copy(ones, cnt_acc.at[dstb.at[slot, j]],
                                     sem_c, add=True)

        def _drain_scatters(slot, rp):
            for j in range(KB):
                pltpu.make_async_copy(rows.at[rp, pl.ds(j * CH, CH)],
                                      acc.at[dstb.at[slot, j]], sem_s).wait()
                if with_cnt:
                    pltpu.make_async_copy(ones, cnt_acc.at[dstb.at[slot, j]],
                                          sem_c).wait()

        def _issue_idx(blk, slot):
            pltpu.async_copy(srcs.at[c, gbase + blk], srcb.at[slot], sem_i)
            pltpu.async_copy(dsts.at[gbase + blk], dstb.at[slot], sem_i)

        def _drain_idx():
            pltpu.make_async_copy(srcs.at[c, gbase], srcb.at[0], sem_i).wait()
            pltpu.make_async_copy(dsts.at[gbase], dstb.at[0], sem_i).wait()

        pltpu.sync_copy(srcs.at[c, gbase], srcb.at[0])
        pltpu.sync_copy(dsts.at[gbase], dstb.at[0])
        _fire_gathers(0, 0)
        _issue_idx(1, 1)

        def outer(bb, carry):
            for par in range(4):
                blk = bb * 4 + par
                rp, rq = par % 2, 1 - par % 2
                nslot = (par + 1) % 4
                pslot = (par + 2) % 4

                @pl.when(blk < BLOCKS - 1)
                def _():
                    _drain_idx()

                @pl.when(blk >= 1)
                def _():
                    _drain_scatters((par + 3) % 4, rq)

                @pl.when(blk < BLOCKS - 1)
                def _():
                    _fire_gathers(nslot, rq)

                @pl.when(blk < BLOCKS - 2)
                def _():
                    _issue_idx(blk + 2, pslot)

                _drain_gathers(par, rp)
                _issue_scatters(par, rp)
            return carry
        lax.fori_loop(0, BLOCKS // 4, outer, 0)
        _drain_scatters((BLOCKS - 1) % 4, (BLOCKS - 1) % 2)
        plsc.subcore_barrier()

        @pl.when(s < N_SUB - 1)
        def _():
            pltpu.sync_copy(acc.at[pl.ds(base0, WB)],
                            out.at[c, pl.ds(base0, WB)])
            if with_cnt:
                pltpu.sync_copy(cnt_acc.at[pl.ds(base0, WB)],
                                cnt_out.at[c, pl.ds(base0, WB)])

        @pl.when(s == N_SUB - 1)
        def _():
            nlast = N - (N_SUB - 1) * WB
            pltpu.sync_copy(acc.at[pl.ds((N_SUB - 1) * WB, nlast)],
                            out.at[c, pl.ds((N_SUB - 1) * WB, nlast)])
            if with_cnt:
                pltpu.sync_copy(cnt_acc.at[pl.ds((N_SUB - 1) * WB, nlast)],
                                cnt_out.at[c, pl.ds((N_SUB - 1) * WB, nlast)])

    return spmm


_spmm_cnt = _make_spmm(True)
_spmm_nocnt = _make_spmm(False)


def _prep_edges(src, dst):
    per = E_TOT // N_SUB
    src_r = src.reshape(N_SUB, per)
    pad_src = ((jnp.arange(N_SUB * E_PAD, dtype=jnp.int32) * 97) % N) \
        .reshape(N_SUB, E_PAD)
    srcp = jnp.concatenate([src_r, pad_src], axis=1) \
        .reshape(N_SUB * BLOCKS, KB, CH)
    srcs = jnp.stack([srcp, srcp + N])
    dst_r = dst.reshape(N_SUB, per)
    pad_dst = (N + (jnp.arange(N_SUB * E_PAD, dtype=jnp.int32) % NDUM)) \
        .reshape(N_SUB, E_PAD)
    dstp = jnp.concatenate([dst_r, pad_dst], axis=1) \
        .reshape(N_SUB * BLOCKS, KB, CH)
    return srcs, dstp


def _spmm_sc(x_split, srcs, dsts, with_cnt):
    table = x_split.reshape(2 * N, HH)
    if with_cnt:
        out, cnt = _spmm_cnt(table, srcs, dsts)
        return out, cnt
    (out,) = _spmm_nocnt(table, srcs, dsts)
    return out, None


def kernel(x_column, x_constraint, edge_serves, edge_served_by,
           Wcol, bcol, Wcon, bcon,
           c1_cs_Wl, c1_cs_bl, c1_cs_Wr, c1_sb_Wl, c1_sb_bl, c1_sb_Wr,
           c2_cs_Wl, c2_cs_bl, c2_cs_Wr, c2_sb_Wl, c2_sb_bl, c2_sb_Wr,
           qW1, qb1, qW2, qb2):
    s_cs, d_cs = edge_serves[0], edge_serves[1]
    s_sb, d_sb = edge_served_by[0], edge_served_by[1]

    xc, xn = _proj(x_column, x_constraint, Wcol, bcol, Wcon, bcon)

    cs_srcs, cs_dsts = _prep_edges(s_cs, d_cs)
    sb_srcs, sb_dsts = _prep_edges(s_sb, d_sb)

    sum_cs, cnt_cs = _spmm_sc(xc, cs_srcs, cs_dsts, True)
    sum_sb, cnt_sb = _spmm_sc(xn, sb_srcs, sb_dsts, True)

    xn1 = _conv(sum_cs, cnt_cs, xn, c1_cs_Wl, c1_cs_bl, c1_cs_Wr,
                split_out=True)
    xc1 = _conv(sum_sb, cnt_sb, xc, c1_sb_Wl, c1_sb_bl, c1_sb_Wr)

    sum_sb2, _ = _spmm_sc(xn1, sb_srcs, sb_dsts, False)

    q = _conv(sum_sb2, cnt_sb, xc1, c2_sb_Wl, c2_sb_bl, c2_sb_Wr,
              head_w=(qW1, qb1, qW2, qb2))
    return q.reshape(-1)


# speedup vs baseline: 13.8842x; 1.0236x over previous
import functools

import jax
import jax.numpy as jnp
from jax import lax
from jax.experimental import pallas as pl
from jax.experimental.pallas import tpu as pltpu

N = 50000
H = 64
HH = 32
BN = 1000
GRID = N // BN


def _proj_body(xcol_ref, xcon_ref, wcolT_ref, bcol_ref, wconT_ref, bcon_ref,
               xc_ref, xn_ref):
    xc = jnp.maximum(
        jnp.dot(xcol_ref[...], wcolT_ref[...],
                preferred_element_type=jnp.float32) + bcol_ref[...], 0.0)
    xn = jnp.maximum(
        jnp.dot(xcon_ref[...], wconT_ref[...],
                preferred_element_type=jnp.float32) + bcon_ref[...], 0.0)
    xc_ref[0] = xc[:, :HH]
    xc_ref[1] = xc[:, HH:]
    xn_ref[0] = xn[:, :HH]
    xn_ref[1] = xn[:, HH:]


def _proj(x_column, x_constraint, Wcol, bcol, Wcon, bcon):
    return pl.pallas_call(
        _proj_body,
        grid=(GRID,),
        in_specs=[
            pl.BlockSpec((BN, 128), lambda i: (i, 0)),
            pl.BlockSpec((BN, 64), lambda i: (i, 0)),
            pl.BlockSpec((128, 64), lambda i: (0, 0)),
            pl.BlockSpec((1, 64), lambda i: (0, 0)),
            pl.BlockSpec((64, 64), lambda i: (0, 0)),
            pl.BlockSpec((1, 64), lambda i: (0, 0)),
        ],
        out_specs=[
            pl.BlockSpec((2, BN, HH), lambda i: (0, i, 0)),
            pl.BlockSpec((2, BN, HH), lambda i: (0, i, 0)),
        ],
        out_shape=[
            jax.ShapeDtypeStruct((2, N, HH), jnp.float32),
            jax.ShapeDtypeStruct((2, N, HH), jnp.float32),
        ],
    )(x_column, x_constraint, Wcol.T, bcol.reshape(1, 64),
      Wcon.T, bcon.reshape(1, 64))


def _conv_body(split_out, head, *refs):
    if head:
        (s_ref, xd_ref, wlT_ref, bl_ref, wrT_ref,
         qW1T_ref, qb1_ref, qW2T_ref, qb2_ref, out_ref) = refs
    else:
        s_ref, xd_ref, wlT_ref, bl_ref, wrT_ref, out_ref = refs
    mean = jnp.concatenate([s_ref[0], s_ref[1]], axis=1)
    if xd_ref.shape[0] == 2:
        xd = jnp.concatenate([xd_ref[0], xd_ref[1]], axis=1)
    else:
        xd = xd_ref[...]
    out = jnp.maximum(
        jnp.dot(mean, wlT_ref[...], preferred_element_type=jnp.float32)
        + bl_ref[...]
        + jnp.dot(xd, wrT_ref[...], preferred_element_type=jnp.float32), 0.0)
    if head:
        h = jnp.maximum(
            jnp.dot(out, qW1T_ref[...], preferred_element_type=jnp.float32)
            + qb1_ref[...], 0.0)
        q = jnp.dot(h, qW2T_ref[...], preferred_element_type=jnp.float32) \
            + qb2_ref[...]
        out_ref[...] = q
    elif split_out:
        out_ref[0] = out[:, :HH]
        out_ref[1] = out[:, HH:]
    else:
        out_ref[...] = out


def _conv(summed, xd, Wl, bl, Wr, split_out=False, head_w=None):
    head = head_w is not None
    xd_split = (xd.ndim == 3)
    in_specs = [
        pl.BlockSpec((2, BN, HH), lambda i: (0, i, 0)),
        (pl.BlockSpec((2, BN, HH), lambda i: (0, i, 0)) if xd_split
         else pl.BlockSpec((BN, 64), lambda i: (i, 0))),
        pl.BlockSpec((64, 64), lambda i: (0, 0)),
        pl.BlockSpec((1, 64), lambda i: (0, 0)),
        pl.BlockSpec((64, 64), lambda i: (0, 0)),
    ]
    args = [summed, xd, Wl.T, bl.reshape(1, 64), Wr.T]
    if head:
        qW1, qb1, qW2, qb2 = head_w
        in_specs += [
            pl.BlockSpec((64, 32), lambda i: (0, 0)),
            pl.BlockSpec((1, 32), lambda i: (0, 0)),
            pl.BlockSpec((32, 1), lambda i: (0, 0)),
            pl.BlockSpec((1, 1), lambda i: (0, 0)),
        ]
        args += [qW1.T, qb1.reshape(1, 32), qW2.T, qb2.reshape(1, 1)]
        out_spec = pl.BlockSpec((BN, 1), lambda i: (i, 0))
        out_shape = jax.ShapeDtypeStruct((N, 1), jnp.float32)
    elif split_out:
        out_spec = pl.BlockSpec((2, BN, HH), lambda i: (0, i, 0))
        out_shape = jax.ShapeDtypeStruct((2, N, HH), jnp.float32)
    else:
        out_spec = pl.BlockSpec((BN, 64), lambda i: (i, 0))
        out_shape = jax.ShapeDtypeStruct((N, 64), jnp.float32)
    return pl.pallas_call(
        functools.partial(_conv_body, split_out, head),
        grid=(GRID,),
        in_specs=in_specs,
        out_specs=out_spec,
        out_shape=out_shape,
    )(*args)


E_TOT = 800000
N_SUB = 16
EPS = 51200
E_PAD = EPS - E_TOT // N_SUB
CH = 128
KB = 2
BLOCKS = EPS // (CH * KB)
NDUM = 400
ACC_ROWS = N + NDUM
ZR = 200
WB = 3200
WCH = 160

from jax.experimental.pallas import tpu_sc as plsc


def _make_spmm():
    with_cnt = True
    mesh = plsc.VectorSubcoreMesh(core_axis_name="c", subcore_axis_name="s")
    out_type = [jax.ShapeDtypeStruct((2, N, HH), jnp.float32)]
    scratch = [
        pltpu.VMEM_SHARED((ACC_ROWS, HH), jnp.float32),
        pltpu.VMEM_SHARED((ACC_ROWS,), jnp.float32),
        pltpu.VMEM((4, KB, CH), jnp.int32),
        pltpu.VMEM((4, KB, CH), jnp.int32),
        pltpu.VMEM((2, KB * CH, HH), jnp.float32),
        pltpu.SemaphoreType.DMA,
        pltpu.SemaphoreType.DMA,
        pltpu.SemaphoreType.DMA,
        pltpu.SemaphoreType.DMA,
        pltpu.VMEM((CH,), jnp.float32),
        pltpu.VMEM((ZR,), jnp.float32),
        pltpu.VMEM((WCH,), jnp.float32),
        pltpu.SemaphoreType.DMA,
    ]

    @functools.partial(
        pl.kernel, mesh=mesh, out_type=out_type, scratch_types=scratch,
        compiler_params=pltpu.CompilerParams(use_tc_tiling_on_sc=False))
    def spmm(table, srcs, dsts, *rest):
        (out, acc, cnt_acc, srcb, dstb, rows, sem_g0, sem_g1,
         sem_s, sem_i, ones, zrow1, cntbuf, sem_c) = rest
        sem_g = (sem_g0, sem_g1)
        c = lax.axis_index("c")
        s = lax.axis_index("s")

        def zbody(i, carry):
            z16 = jnp.zeros((16,), jnp.float32)
            rows[0, i, pl.ds(0, 16)] = z16
            rows[0, i, pl.ds(16, 16)] = z16
            return carry
        lax.fori_loop(0, ZR, zbody, 0)
        if with_cnt:
            def z1body(i, carry):
                zrow1[pl.ds(i * 16, 16)] = jnp.zeros((16,), jnp.float32)
                return carry
            lax.fori_loop(0, ZR // 16, z1body, 0)
            for k in range(CH // 16):
                ones[pl.ds(k * 16, 16)] = jnp.ones((16,), jnp.float32)

        base0 = s * WB
        ncop = jnp.where(s < N_SUB - 1, WB // ZR, (ACC_ROWS - 15 * WB) // ZR)
        def zacc(kk, carry):
            off = base0 + kk * ZR
            pltpu.sync_copy(rows.at[0, pl.ds(0, ZR)], acc.at[pl.ds(off, ZR)])
            if with_cnt:
                pltpu.sync_copy(zrow1, cnt_acc.at[pl.ds(off, ZR)])
            return carry
        lax.fori_loop(0, ncop, zacc, 0)
        plsc.subcore_barrier()

        gbase = s * BLOCKS

        def _fire_gathers(slot, rp):
            for j in range(KB):
                pltpu.async_copy(table.at[srcb.at[slot, j]],
                                 rows.at[rp, pl.ds(j * CH, CH)], sem_g[rp])

        def _drain_gathers(slot, rp):
            for j in range(KB):
                pltpu.make_async_copy(
                    table.at[srcb.at[slot, j]],
                    rows.at[rp, pl.ds(j * CH, CH)], sem_g[rp]).wait()

        def _issue_scatters(slot, rp):
            for j in range(KB):
                pltpu.async_copy(rows.at[rp, pl.ds(j * CH, CH)],
                                 acc.at[dstb.at[slot, j]], sem_s, add=True)
                if with_cnt:
                    pltpu.async_copy(ones, cnt_acc.at[dstb.at[slot, j]],
                                     sem_c, add=True)

        def _drain_scatters(slot, rp):
            for j in range(KB):
                pltpu.make_async_copy(rows.at[rp, pl.ds(j * CH, CH)],
                                      acc.at[dstb.at[slot, j]], sem_s).wait()
                if with_cnt:
                    pltpu.make_async_copy(ones, cnt_acc.at[dstb.at[slot, j]],
                                          sem_c).wait()

        def _issue_idx(blk, slot):
            pltpu.async_copy(srcs.at[c, gbase + blk], srcb.at[slot], sem_i)
            pltpu.async_copy(dsts.at[gbase + blk], dstb.at[slot], sem_i)

        def _drain_idx():
            pltpu.make_async_copy(srcs.at[c, gbase], srcb.at[0], sem_i).wait()
            pltpu.make_async_copy(dsts.at[gbase], dstb.at[0], sem_i).wait()

        pltpu.sync_copy(srcs.at[c, gbase], srcb.at[0])
        pltpu.sync_copy(dsts.at[gbase], dstb.at[0])
        _fire_gathers(0, 0)
        _issue_idx(1, 1)

        def outer(bb, carry):
            for par in range(4):
                blk = bb * 4 + par
                rp, rq = par % 2, 1 - par % 2
                nslot = (par + 1) % 4
                pslot = (par + 2) % 4

                @pl.when(blk < BLOCKS - 1)
                def _():
                    _drain_idx()

                @pl.when(blk >= 1)
                def _():
                    _drain_scatters((par + 3) % 4, rq)

                @pl.when(blk < BLOCKS - 1)
                def _():
                    _fire_gathers(nslot, rq)

                @pl.when(blk < BLOCKS - 2)
                def _():
                    _issue_idx(blk + 2, pslot)

                _drain_gathers(par, rp)
                _issue_scatters(par, rp)
            return carry
        lax.fori_loop(0, BLOCKS // 4, outer, 0)
        _drain_scatters((BLOCKS - 1) % 4, (BLOCKS - 1) % 2)
        plsc.subcore_barrier()

        def wb_chunk(off, nrows):
            pltpu.sync_copy(acc.at[pl.ds(off, nrows)],
                            rows.at[0, pl.ds(0, nrows)])
            pltpu.sync_copy(cnt_acc.at[pl.ds(off, nrows)],
                            cntbuf.at[pl.ds(0, nrows)])

            def grp(g, carry2):
                cv = cntbuf[pl.ds(g * 16, 16)]
                inv = 1.0 / jnp.maximum(cv, 1.0)
                for j in range(16):
                    r = g * 16 + j
                    iv = inv[j]
                    rows[0, r, pl.ds(0, 16)] = rows[0, r, pl.ds(0, 16)] * iv
                    rows[0, r, pl.ds(16, 16)] = rows[0, r, pl.ds(16, 16)] * iv
                return carry2
            lax.fori_loop(0, nrows // 16, grp, 0)
            pltpu.sync_copy(rows.at[0, pl.ds(0, nrows)],
                            out.at[c, pl.ds(off, nrows)])

        nch = jnp.where(s < N_SUB - 1, WB // WCH, 12)

        def wb_body(k, carry):
            wb_chunk(base0 + k * WCH, WCH)
            return carry
        lax.fori_loop(0, nch, wb_body, 0)

        @pl.when(s == N_SUB - 1)
        def _():
            wb_chunk((N_SUB - 1) * WB + 12 * WCH, 80)

    return spmm


_spmm_kernel = _make_spmm()


def _prep_edges(src, dst):
    per = E_TOT // N_SUB
    src_r = src.reshape(N_SUB, per)
    pad_src = ((jnp.arange(N_SUB * E_PAD, dtype=jnp.int32) * 97) % N) \
        .reshape(N_SUB, E_PAD)
    srcp = jnp.concatenate([src_r, pad_src], axis=1) \
        .reshape(N_SUB * BLOCKS, KB, CH)
    srcs = jnp.stack([srcp, srcp + N])
    dst_r = dst.reshape(N_SUB, per)
    pad_dst = (N + (jnp.arange(N_SUB * E_PAD, dtype=jnp.int32) % NDUM)) \
        .reshape(N_SUB, E_PAD)
    dstp = jnp.concatenate([dst_r, pad_dst], axis=1) \
        .reshape(N_SUB * BLOCKS, KB, CH)
    return srcs, dstp


def _spmm_sc(x_split, srcs, dsts):
    table = x_split.reshape(2 * N, HH)
    (out,) = _spmm_kernel(table, srcs, dsts)
    return out


def kernel(x_column, x_constraint, edge_serves, edge_served_by,
           Wcol, bcol, Wcon, bcon,
           c1_cs_Wl, c1_cs_bl, c1_cs_Wr, c1_sb_Wl, c1_sb_bl, c1_sb_Wr,
           c2_cs_Wl, c2_cs_bl, c2_cs_Wr, c2_sb_Wl, c2_sb_bl, c2_sb_Wr,
           qW1, qb1, qW2, qb2):
    s_cs, d_cs = edge_serves[0], edge_serves[1]
    s_sb, d_sb = edge_served_by[0], edge_served_by[1]

    xc, xn = _proj(x_column, x_constraint, Wcol, bcol, Wcon, bcon)

    cs_srcs, cs_dsts = _prep_edges(s_cs, d_cs)
    sb_srcs, sb_dsts = _prep_edges(s_sb, d_sb)

    mean_cs = _spmm_sc(xc, cs_srcs, cs_dsts)
    mean_sb = _spmm_sc(xn, sb_srcs, sb_dsts)

    xn1 = _conv(mean_cs, xn, c1_cs_Wl, c1_cs_bl, c1_cs_Wr, split_out=True)
    xc1 = _conv(mean_sb, xc, c1_sb_Wl, c1_sb_bl, c1_sb_Wr)

    mean_sb2 = _spmm_sc(xn1, sb_srcs, sb_dsts)

    q = _conv(mean_sb2, xc1, c2_sb_Wl, c2_sb_bl, c2_sb_Wr,
              head_w=(qW1, qb1, qW2, qb2))
    return q.reshape(-1)
